# Initial kernel scaffold; baseline (speedup 1.0000x reference)
import jax
import jax.numpy as jnp
from jax.experimental import pallas as pl


def kernel(x, edges_idx, batch_idx, g_features, W1, b1, W2, b2, Wg, bg, Wout, bout):
    raise NotImplementedError("write your pallas kernel here")


# speedup vs baseline: 13.8190x; 13.8190x over previous
import functools

import jax
import jax.numpy as jnp
from jax import lax
from jax.experimental import pallas as pl
from jax.experimental.pallas import tpu as pltpu
from jax.experimental.pallas import tpu_sc as plsc

N = 10000
E = 320000
D = 128
NG = 64
NC = 2
NS = 16
K = 80
CW = 16
EPC = E // NC
EPT = EPC // NS
ITS = EPT // K
RPT = 624
RTAIL = N - NS * RPT

_MESH = dict(core_axis_name="c", subcore_axis_name="s")


KD = 2000
RLEN = RPT + RTAIL


@functools.partial(
    pl.kernel,
    out_type=jax.ShapeDtypeStruct((NC * N,), jnp.float32),
    mesh=plsc.VectorSubcoreMesh(**_MESH),
    compiler_params=pltpu.CompilerParams(needs_layout_passes=False),
    scratch_types=[
        pltpu.VMEM_SHARED((NS * N,), jnp.float32),
        pltpu.VMEM((N,), jnp.float32),
        pltpu.VMEM((KD,), jnp.int32),
        pltpu.VMEM((NS * RLEN,), jnp.float32),
    ],
)
def _deg_kernel(dst_hbm, out_hbm, shared, hist, dst_v, red_v):
    c = lax.axis_index("c")
    s = lax.axis_index("s")

    @pl.loop(0, N // 16)
    def _(i):
        hist[pl.ds(i * 16, 16)] = jnp.zeros((16,), jnp.float32)

    base = c * EPC + s * EPT
    ones16 = jnp.full((16,), 1.0, jnp.float32)

    @pl.loop(0, EPT // KD)
    def _(i):
        pltpu.sync_copy(dst_hbm.at[pl.ds(base + i * KD, KD)], dst_v)

        @pl.loop(0, KD // 16)
        def _(j):
            idx = dst_v[pl.ds(j * 16, 16)]
            plsc.addupdate_scatter(hist, [idx], ones16)

    pltpu.sync_copy(hist, shared.at[pl.ds(s * N, N)])
    plsc.subcore_barrier()

    n0 = s * RPT
    for p in range(NS):
        pltpu.sync_copy(shared.at[pl.ds(p * N + n0, RLEN)],
                        red_v.at[pl.ds(p * RLEN, RLEN)])

    @pl.loop(0, RLEN // 16)
    def _(j):
        acc = jnp.zeros((16,), jnp.float32)
        for p in range(NS):
            acc = acc + red_v[pl.ds(p * RLEN + j * 16, 16)]
        hist[pl.ds(j * 16, 16)] = acc

    pltpu.sync_copy(hist.at[pl.ds(0, RPT)], out_hbm.at[pl.ds(c * N + n0, RPT)])

    @pl.when(s == NS - 1)
    def _():
        pltpu.sync_copy(hist.at[pl.ds(RPT, RTAIL)],
                        out_hbm.at[pl.ds(c * N + n0 + RPT, RTAIL)])


@functools.partial(
    pl.kernel,
    out_type=jax.ShapeDtypeStruct((NC, N, D), jnp.float32),
    mesh=plsc.VectorSubcoreMesh(**_MESH),
    scratch_types=[
        pltpu.VMEM_SHARED((N, D), jnp.float32),
        pltpu.VMEM((K,), jnp.int32),
        pltpu.VMEM((K,), jnp.int32),
        pltpu.VMEM((K, D), jnp.float32),
        pltpu.SemaphoreType.DMA,
    ],
)
def _edge_kernel(src_hbm, dst_hbm, ht_hbm, zeros_hbm, out_hbm,
                 acc, src_v, dst_v, rows_v, sem):
    c = lax.axis_index("c")
    s = lax.axis_index("s")
    r0 = s * RPT
    pltpu.sync_copy(zeros_hbm.at[pl.ds(r0, RPT)], acc.at[pl.ds(r0, RPT)])

    @pl.when(s == 0)
    def _():
        pltpu.sync_copy(zeros_hbm.at[pl.ds(NS * RPT, RTAIL)],
                        acc.at[pl.ds(NS * RPT, RTAIL)])

    plsc.subcore_barrier()
    base = c * EPC + s * EPT

    @pl.loop(0, ITS)
    def _(i):
        e0 = base + i * K
        pltpu.sync_copy(src_hbm.at[pl.ds(e0, K)], src_v)
        pltpu.sync_copy(dst_hbm.at[pl.ds(e0, K)], dst_v)
        pltpu.async_copy(ht_hbm.at[src_v], rows_v, sem).wait()
        pltpu.sync_copy(rows_v, acc.at[dst_v], add=True)

    plsc.subcore_barrier()
    pltpu.sync_copy(acc.at[pl.ds(r0, RPT)], out_hbm.at[c, pl.ds(r0, RPT)])

    @pl.when(s == 0)
    def _():
        pltpu.sync_copy(acc.at[pl.ds(NS * RPT, RTAIL)],
                        out_hbm.at[c, pl.ds(NS * RPT, RTAIL)])


_RB = 1000


def _dinv_of(d0_ref, d1_ref):
    deg = 1.0 + d0_ref[...] + d1_ref[...]
    return lax.rsqrt(deg)


def _ht1_body(x_ref, d0_ref, d1_ref, w_ref, o_ref):
    h = jnp.dot(x_ref[...], w_ref[...], preferred_element_type=jnp.float32)
    o_ref[...] = h * _dinv_of(d0_ref, d1_ref)


_ht1_call = pl.pallas_call(
    _ht1_body,
    grid=(N // _RB,),
    in_specs=[
        pl.BlockSpec((_RB, D), lambda i: (i, 0)),
        pl.BlockSpec((_RB, 1), lambda i: (i, 0)),
        pl.BlockSpec((_RB, 1), lambda i: (i, 0)),
        pl.BlockSpec((D, D), lambda i: (0, 0)),
    ],
    out_specs=pl.BlockSpec((_RB, D), lambda i: (i, 0)),
    out_shape=jax.ShapeDtypeStruct((N, D), jnp.float32),
)


def _ht2_body(acc_ref, ht1_ref, d0_ref, d1_ref, w_ref, b_ref, o_ref):
    dinv = _dinv_of(d0_ref, d1_ref)
    out1 = dinv * (ht1_ref[...] + acc_ref[0] + acc_ref[1]) + b_ref[...]
    o_ref[...] = jnp.dot(out1, w_ref[...], preferred_element_type=jnp.float32) * dinv


_ht2_call = pl.pallas_call(
    _ht2_body,
    grid=(N // _RB,),
    in_specs=[
        pl.BlockSpec((NC, _RB, D), lambda i: (0, i, 0)),
        pl.BlockSpec((_RB, D), lambda i: (i, 0)),
        pl.BlockSpec((_RB, 1), lambda i: (i, 0)),
        pl.BlockSpec((_RB, 1), lambda i: (i, 0)),
        pl.BlockSpec((D, D), lambda i: (0, 0)),
        pl.BlockSpec((1, D), lambda i: (0, 0)),
    ],
    out_specs=pl.BlockSpec((_RB, D), lambda i: (i, 0)),
    out_shape=jax.ShapeDtypeStruct((N, D), jnp.float32),
)


_PB = 400
_NPB = N // _PB


def _head_body(acc_ref, ht2_ref, d0_ref, d1_ref, b2_ref, bi_ref, gf_ref,
               wg_ref, bg_ref, wo_ref, bo_ref, o_ref, sum_s, max_s, cnt_s):
    i = pl.program_id(0)

    @pl.when(i == 0)
    def _():
        sum_s[...] = jnp.zeros_like(sum_s)
        cnt_s[...] = jnp.zeros_like(cnt_s)
        max_s[...] = jnp.full_like(max_s, -jnp.inf)

    dinv = _dinv_of(d0_ref, d1_ref)
    h2 = dinv * (ht2_ref[...] + acc_ref[0] + acc_ref[1]) + b2_ref[...]

    b = bi_ref[0, 0, :]
    bc = b[:, None]
    oh = (bc == lax.broadcasted_iota(jnp.int32, (_PB, NG), 1)).astype(jnp.float32)
    sum_s[...] += lax.dot_general(oh, h2, (((0,), (0,)), ((), ())),
                                  preferred_element_type=jnp.float32)
    cnt_s[...] += jnp.sum(oh, axis=0)[:, None]

    gmin = jnp.min(b)
    gmax = jnp.max(b)
    for g in range(NG):
        @pl.when((gmin <= g) & (g <= gmax))
        def _():
            m = jnp.max(jnp.where(bc == g, h2, -jnp.inf), axis=0, keepdims=True)
            max_s[g:g + 1, :] = jnp.maximum(max_s[g:g + 1, :], m)

    @pl.when(i == _NPB - 1)
    def _():
        cnt = cnt_s[...]
        mean = sum_s[...] / jnp.maximum(cnt, 1.0)
        mx = jnp.where(cnt > 0.0, max_s[...], 0.0)
        gft = jnp.dot(gf_ref[...], wg_ref[...],
                      preferred_element_type=jnp.float32) + bg_ref[...]
        flat = jnp.concatenate([mean, mx, gft], axis=1)
        logits = jnp.dot(flat, wo_ref[...],
                         preferred_element_type=jnp.float32) + bo_ref[...]
        m0 = jnp.max(logits, axis=1, keepdims=True)
        lse = jnp.log(jnp.sum(jnp.exp(logits - m0), axis=1, keepdims=True)) + m0
        o_ref[...] = logits - lse


_head_call = pl.pallas_call(
    _head_body,
    grid=(_NPB,),
    in_specs=[
        pl.BlockSpec((NC, _PB, D), lambda i: (0, i, 0)),
        pl.BlockSpec((_PB, D), lambda i: (i, 0)),
        pl.BlockSpec((_PB, 1), lambda i: (i, 0)),
        pl.BlockSpec((_PB, 1), lambda i: (i, 0)),
        pl.BlockSpec((1, D), lambda i: (0, 0)),
        pl.BlockSpec((1, 1, _PB), lambda i: (i, 0, 0)),
        pl.BlockSpec((NG, 16), lambda i: (0, 0)),
        pl.BlockSpec((16, D), lambda i: (0, 0)),
        pl.BlockSpec((1, D), lambda i: (0, 0)),
        pl.BlockSpec((3 * D, 2), lambda i: (0, 0)),
        pl.BlockSpec((1, 2), lambda i: (0, 0)),
    ],
    out_specs=pl.BlockSpec((NG, 2), lambda i: (0, 0)),
    out_shape=jax.ShapeDtypeStruct((NG, 2), jnp.float32),
    scratch_shapes=[
        pltpu.VMEM((NG, D), jnp.float32),
        pltpu.VMEM((NG, D), jnp.float32),
        pltpu.VMEM((NG, D), jnp.float32),
    ],
)


def kernel(x, edges_idx, batch_idx, g_features, W1, b1, W2, b2, Wg, bg,
           Wout, bout):
    src = edges_idx[0]
    dst = edges_idx[1]
    z_feat = jnp.zeros((N, D), jnp.float32)

    degf = _deg_kernel(dst)
    d0 = degf[:N].reshape(N, 1)
    d1 = degf[N:].reshape(N, 1)
    ht1 = _ht1_call(x, d0, d1, W1)
    acc1 = _edge_kernel(src, dst, ht1, z_feat)
    ht2 = _ht2_call(acc1, ht1, d0, d1, W2, b1.reshape(1, D))
    acc2 = _edge_kernel(src, dst, ht2, z_feat)
    out = _head_call(acc2, ht2, d0, d1, b2.reshape(1, D),
                     batch_idx.reshape(_NPB, 1, _PB), g_features, Wg,
                     bg.reshape(1, D), Wout, bout.reshape(1, 2))
    return out


# speedup vs baseline: 21.0761x; 1.5252x over previous
import functools

import jax
import jax.numpy as jnp
from jax import lax
from jax.experimental import pallas as pl
from jax.experimental.pallas import tpu as pltpu
from jax.experimental.pallas import tpu_sc as plsc

N = 10000
E = 320000
D = 128
NG = 64
NC = 2
NS = 16
K = 80
CW = 16
EPC = E // NC
EPT = EPC // NS
ITS = EPT // K
RPT = 624
RTAIL = N - NS * RPT

_MESH = dict(core_axis_name="c", subcore_axis_name="s")


KD = 2000
RLEN = RPT + RTAIL


@functools.partial(
    pl.kernel,
    out_type=jax.ShapeDtypeStruct((NC * N,), jnp.float32),
    mesh=plsc.VectorSubcoreMesh(**_MESH),
    compiler_params=pltpu.CompilerParams(needs_layout_passes=False),
    scratch_types=[
        pltpu.VMEM_SHARED((NS * N,), jnp.float32),
        pltpu.VMEM((N,), jnp.float32),
        pltpu.VMEM((KD,), jnp.int32),
        pltpu.VMEM((NS * RLEN,), jnp.float32),
    ],
)
def _deg_kernel(dst_hbm, out_hbm, shared, hist, dst_v, red_v):
    c = lax.axis_index("c")
    s = lax.axis_index("s")

    @pl.loop(0, N // 16)
    def _(i):
        hist[pl.ds(i * 16, 16)] = jnp.zeros((16,), jnp.float32)

    base = c * EPC + s * EPT
    ones16 = jnp.full((16,), 1.0, jnp.float32)

    @pl.loop(0, EPT // KD)
    def _(i):
        pltpu.sync_copy(dst_hbm.at[pl.ds(base + i * KD, KD)], dst_v)

        @pl.loop(0, KD // 16)
        def _(j):
            idx = dst_v[pl.ds(j * 16, 16)]
            plsc.addupdate_scatter(hist, [idx], ones16)

    pltpu.sync_copy(hist, shared.at[pl.ds(s * N, N)])
    plsc.subcore_barrier()

    n0 = s * RPT
    for p in range(NS):
        pltpu.sync_copy(shared.at[pl.ds(p * N + n0, RLEN)],
                        red_v.at[pl.ds(p * RLEN, RLEN)])

    @pl.loop(0, RLEN // 16)
    def _(j):
        acc = jnp.zeros((16,), jnp.float32)
        for p in range(NS):
            acc = acc + red_v[pl.ds(p * RLEN + j * 16, 16)]
        hist[pl.ds(j * 16, 16)] = acc

    pltpu.sync_copy(hist.at[pl.ds(0, RPT)], out_hbm.at[pl.ds(c * N + n0, RPT)])

    @pl.when(s == NS - 1)
    def _():
        pltpu.sync_copy(hist.at[pl.ds(RPT, RTAIL)],
                        out_hbm.at[pl.ds(c * N + n0 + RPT, RTAIL)])


@functools.partial(
    pl.kernel,
    out_type=jax.ShapeDtypeStruct((NC, N, D), jnp.float32),
    mesh=plsc.VectorSubcoreMesh(**_MESH),
    scratch_types=[
        pltpu.VMEM_SHARED((N, D), jnp.float32),
        pltpu.VMEM((K,), jnp.int32),
        pltpu.VMEM((K,), jnp.int32),
        pltpu.VMEM((K,), jnp.int32),
        pltpu.VMEM((K,), jnp.int32),
        pltpu.VMEM((K, D), jnp.float32),
        pltpu.VMEM((K, D), jnp.float32),
        pltpu.SemaphoreType.DMA,
        pltpu.SemaphoreType.DMA,
    ],
)
def _edge_kernel(src_hbm, dst_hbm, ht_hbm, zeros_hbm, out_hbm,
                 acc, src_v0, dst_v0, src_v1, dst_v1, rows0, rows1,
                 sem0, sem1):
    c = lax.axis_index("c")
    s = lax.axis_index("s")
    r0 = s * RPT
    pltpu.sync_copy(zeros_hbm.at[pl.ds(r0, RPT)], acc.at[pl.ds(r0, RPT)])

    @pl.when(s == 0)
    def _():
        pltpu.sync_copy(zeros_hbm.at[pl.ds(NS * RPT, RTAIL)],
                        acc.at[pl.ds(NS * RPT, RTAIL)])

    plsc.subcore_barrier()
    base = c * EPC + s * EPT

    pltpu.sync_copy(src_hbm.at[pl.ds(base, K)], src_v0)
    pltpu.sync_copy(dst_hbm.at[pl.ds(base, K)], dst_v0)
    pltpu.async_copy(ht_hbm.at[src_v0], rows0, sem0)

    @pl.loop(0, (ITS - 1) // 2)
    def _(p):
        eb = base + (2 * p + 1) * K
        pltpu.sync_copy(src_hbm.at[pl.ds(eb, K)], src_v1)
        pltpu.sync_copy(dst_hbm.at[pl.ds(eb, K)], dst_v1)
        pltpu.async_copy(ht_hbm.at[src_v1], rows1, sem1)
        pltpu.make_async_copy(ht_hbm.at[src_v0], rows0, sem0).wait()
        pltpu.sync_copy(rows0, acc.at[dst_v0], add=True)
        ea = eb + K
        pltpu.sync_copy(src_hbm.at[pl.ds(ea, K)], src_v0)
        pltpu.sync_copy(dst_hbm.at[pl.ds(ea, K)], dst_v0)
        pltpu.async_copy(ht_hbm.at[src_v0], rows0, sem0)
        pltpu.make_async_copy(ht_hbm.at[src_v1], rows1, sem1).wait()
        pltpu.sync_copy(rows1, acc.at[dst_v1], add=True)

    pltpu.make_async_copy(ht_hbm.at[src_v0], rows0, sem0).wait()
    pltpu.sync_copy(rows0, acc.at[dst_v0], add=True)

    plsc.subcore_barrier()
    pltpu.sync_copy(acc.at[pl.ds(r0, RPT)], out_hbm.at[c, pl.ds(r0, RPT)])

    @pl.when(s == 0)
    def _():
        pltpu.sync_copy(acc.at[pl.ds(NS * RPT, RTAIL)],
                        out_hbm.at[c, pl.ds(NS * RPT, RTAIL)])


_RB = 1000


def _dinv_of(d0_ref, d1_ref):
    deg = 1.0 + d0_ref[...] + d1_ref[...]
    return lax.rsqrt(deg)


def _ht1_body(x_ref, d0_ref, d1_ref, w_ref, o_ref):
    h = jnp.dot(x_ref[...], w_ref[...], preferred_element_type=jnp.float32)
    o_ref[...] = h * _dinv_of(d0_ref, d1_ref)


_ht1_call = pl.pallas_call(
    _ht1_body,
    grid=(N // _RB,),
    in_specs=[
        pl.BlockSpec((_RB, D), lambda i: (i, 0)),
        pl.BlockSpec((_RB, 1), lambda i: (i, 0)),
        pl.BlockSpec((_RB, 1), lambda i: (i, 0)),
        pl.BlockSpec((D, D), lambda i: (0, 0)),
    ],
    out_specs=pl.BlockSpec((_RB, D), lambda i: (i, 0)),
    out_shape=jax.ShapeDtypeStruct((N, D), jnp.float32),
)


def _ht2_body(acc_ref, ht1_ref, d0_ref, d1_ref, w_ref, b_ref, o_ref):
    dinv = _dinv_of(d0_ref, d1_ref)
    out1 = dinv * (ht1_ref[...] + acc_ref[0] + acc_ref[1]) + b_ref[...]
    o_ref[...] = jnp.dot(out1, w_ref[...], preferred_element_type=jnp.float32) * dinv


_ht2_call = pl.pallas_call(
    _ht2_body,
    grid=(N // _RB,),
    in_specs=[
        pl.BlockSpec((NC, _RB, D), lambda i: (0, i, 0)),
        pl.BlockSpec((_RB, D), lambda i: (i, 0)),
        pl.BlockSpec((_RB, 1), lambda i: (i, 0)),
        pl.BlockSpec((_RB, 1), lambda i: (i, 0)),
        pl.BlockSpec((D, D), lambda i: (0, 0)),
        pl.BlockSpec((1, D), lambda i: (0, 0)),
    ],
    out_specs=pl.BlockSpec((_RB, D), lambda i: (i, 0)),
    out_shape=jax.ShapeDtypeStruct((N, D), jnp.float32),
)


_PB = 400
_NPB = N // _PB


def _head_body(acc_ref, ht2_ref, d0_ref, d1_ref, b2_ref, bi_ref, gf_ref,
               wg_ref, bg_ref, wo_ref, bo_ref, o_ref, sum_s, max_s, cnt_s):
    i = pl.program_id(0)

    @pl.when(i == 0)
    def _():
        sum_s[...] = jnp.zeros_like(sum_s)
        cnt_s[...] = jnp.zeros_like(cnt_s)
        max_s[...] = jnp.full_like(max_s, -jnp.inf)

    dinv = _dinv_of(d0_ref, d1_ref)
    h2 = dinv * (ht2_ref[...] + acc_ref[0] + acc_ref[1]) + b2_ref[...]

    b = bi_ref[0, 0, :]
    bc = b[:, None]
    oh = (bc == lax.broadcasted_iota(jnp.int32, (_PB, NG), 1)).astype(jnp.float32)
    sum_s[...] += lax.dot_general(oh, h2, (((0,), (0,)), ((), ())),
                                  preferred_element_type=jnp.float32)
    cnt_s[...] += jnp.sum(oh, axis=0)[:, None]

    gmin = jnp.min(b)
    gmax = jnp.max(b)
    for g in range(NG):
        @pl.when((gmin <= g) & (g <= gmax))
        def _():
            m = jnp.max(jnp.where(bc == g, h2, -jnp.inf), axis=0, keepdims=True)
            max_s[g:g + 1, :] = jnp.maximum(max_s[g:g + 1, :], m)

    @pl.when(i == _NPB - 1)
    def _():
        cnt = cnt_s[...]
        mean = sum_s[...] / jnp.maximum(cnt, 1.0)
        mx = jnp.where(cnt > 0.0, max_s[...], 0.0)
        gft = jnp.dot(gf_ref[...], wg_ref[...],
                      preferred_element_type=jnp.float32) + bg_ref[...]
        flat = jnp.concatenate([mean, mx, gft], axis=1)
        logits = jnp.dot(flat, wo_ref[...],
                         preferred_element_type=jnp.float32) + bo_ref[...]
        m0 = jnp.max(logits, axis=1, keepdims=True)
        lse = jnp.log(jnp.sum(jnp.exp(logits - m0), axis=1, keepdims=True)) + m0
        o_ref[...] = logits - lse


_head_call = pl.pallas_call(
    _head_body,
    grid=(_NPB,),
    in_specs=[
        pl.BlockSpec((NC, _PB, D), lambda i: (0, i, 0)),
        pl.BlockSpec((_PB, D), lambda i: (i, 0)),
        pl.BlockSpec((_PB, 1), lambda i: (i, 0)),
        pl.BlockSpec((_PB, 1), lambda i: (i, 0)),
        pl.BlockSpec((1, D), lambda i: (0, 0)),
        pl.BlockSpec((1, 1, _PB), lambda i: (i, 0, 0)),
        pl.BlockSpec((NG, 16), lambda i: (0, 0)),
        pl.BlockSpec((16, D), lambda i: (0, 0)),
        pl.BlockSpec((1, D), lambda i: (0, 0)),
        pl.BlockSpec((3 * D, 2), lambda i: (0, 0)),
        pl.BlockSpec((1, 2), lambda i: (0, 0)),
    ],
    out_specs=pl.BlockSpec((NG, 2), lambda i: (0, 0)),
    out_shape=jax.ShapeDtypeStruct((NG, 2), jnp.float32),
    scratch_shapes=[
        pltpu.VMEM((NG, D), jnp.float32),
        pltpu.VMEM((NG, D), jnp.float32),
        pltpu.VMEM((NG, D), jnp.float32),
    ],
)


def kernel(x, edges_idx, batch_idx, g_features, W1, b1, W2, b2, Wg, bg,
           Wout, bout):
    src = edges_idx[0]
    dst = edges_idx[1]
    z_feat = jnp.zeros((N, D), jnp.float32)

    degf = _deg_kernel(dst)
    d0 = degf[:N].reshape(N, 1)
    d1 = degf[N:].reshape(N, 1)
    ht1 = _ht1_call(x, d0, d1, W1)
    acc1 = _edge_kernel(src, dst, ht1, z_feat)
    ht2 = _ht2_call(acc1, ht1, d0, d1, W2, b1.reshape(1, D))
    acc2 = _edge_kernel(src, dst, ht2, z_feat)
    out = _head_call(acc2, ht2, d0, d1, b2.reshape(1, D),
                     batch_idx.reshape(_NPB, 1, _PB), g_features, Wg,
                     bg.reshape(1, D), Wout, bout.reshape(1, 2))
    return out
